# baseline (device time: 38419 ns/iter reference)
import jax
import jax.numpy as jnp
from jax import lax
from jax.experimental import pallas as pl
from jax.experimental.pallas import tpu as pltpu

N_DEV = 8
N_LAYERS = 3


def kernel(x, Win0, Wout0, Win1, Wout1, Win2, Wout2):
    b, d_shard = x.shape
    h_dim = Win0.shape[1]
    blk = h_dim // N_DEV

    def body(x_ref, win0_ref, wout0_ref, win1_ref, wout1_ref, win2_ref,
             wout2_ref, out_ref, part_ref, rs_ref, ag_ref,
             wf_in, wf_out, wbf_in, wbf_out, w_sems, send_sems, recv_sems):
        my_pos = lax.axis_index("i")

        win_hbm = [win0_ref, win1_ref, win2_ref]
        wout_hbm = [wout0_ref, wout1_ref, wout2_ref]

        win_dmas = [
            pltpu.make_async_copy(win_hbm[l], wf_in.at[l], w_sems.at[0, l])
            for l in range(N_LAYERS)
        ]
        wout_dmas = [
            pltpu.make_async_copy(wout_hbm[l], wf_out.at[l], w_sems.at[1, l])
            for l in range(N_LAYERS)
        ]
        win_dmas[0].start()
        wout_dmas[0].start()
        for l in range(1, N_LAYERS):
            win_dmas[l].start()
            wout_dmas[l].start()

        win_dmas[0].wait()
        wbf_in[0, :, :] = wf_in[0, :, :].astype(jnp.bfloat16)
        x_cur = x_ref[:, :].astype(jnp.bfloat16)

        all_rdmas = []
        for l in range(N_LAYERS):
            partial = jnp.dot(x_cur, wbf_in[l, :, :],
                              preferred_element_type=jnp.float32)
            for t in range(N_DEV):
                part_ref[l, t, :, :] = (
                    partial[:, t * blk:(t + 1) * blk].astype(jnp.bfloat16))

            if l == 0:
                barrier_sem = pltpu.get_barrier_semaphore()
                for d in range(1, N_DEV):
                    pl.semaphore_signal(
                        barrier_sem, inc=1,
                        device_id=((my_pos + d) % N_DEV,),
                        device_id_type=pl.DeviceIdType.MESH,
                    )
                pl.semaphore_wait(barrier_sem, N_DEV - 1)

            rs_rdmas = []
            for d in range(1, N_DEV):
                t = (my_pos + d) % N_DEV
                rdma = pltpu.make_async_remote_copy(
                    src_ref=part_ref.at[l, t],
                    dst_ref=rs_ref.at[l, d - 1],
                    send_sem=send_sems.at[l, 0, d - 1],
                    recv_sem=recv_sems.at[l, 0, d - 1],
                    device_id=(t,),
                    device_id_type=pl.DeviceIdType.MESH,
                )
                rdma.start()
                rs_rdmas.append(rdma)

            wout_dmas[l].wait()
            wbf_out[l, :, :] = wf_out[l, :, :].astype(jnp.bfloat16)
            if l + 1 < N_LAYERS:
                win_dmas[l + 1].wait()
                wbf_in[l + 1, :, :] = wf_in[l + 1, :, :].astype(jnp.bfloat16)

            acc = part_ref[l, my_pos, :, :].astype(jnp.float32)
            for d in range(1, N_DEV):
                rs_rdmas[d - 1].wait_recv()
                acc = acc + rs_ref[l, d - 1, :, :].astype(jnp.float32)
            hred = jnp.maximum(acc, 0.0).astype(jnp.bfloat16)
            ag_ref[l, my_pos, :, :] = hred

            ag_rdmas = []
            for d in range(1, N_DEV):
                t = (my_pos + d) % N_DEV
                rdma = pltpu.make_async_remote_copy(
                    src_ref=ag_ref.at[l, my_pos],
                    dst_ref=ag_ref.at[l, my_pos],
                    send_sem=send_sems.at[l, 1, d - 1],
                    recv_sem=recv_sems.at[l, 1, d - 1],
                    device_id=(t,),
                    device_id_type=pl.DeviceIdType.MESH,
                )
                rdma.start()
                ag_rdmas.append(rdma)

            nxt = jnp.dot(
                ag_ref[l, my_pos, :, :],
                wbf_out[l, pl.ds(my_pos * blk, blk), :],
                preferred_element_type=jnp.float32)
            for d in range(1, N_DEV):
                ag_rdmas[d - 1].wait_recv()
                s = (my_pos - d) % N_DEV
                nxt = nxt + jnp.dot(
                    ag_ref[l, s, :, :],
                    wbf_out[l, pl.ds(s * blk, blk), :],
                    preferred_element_type=jnp.float32)
            if l == N_LAYERS - 1:
                out_ref[:, :] = nxt
            else:
                x_cur = nxt.astype(jnp.bfloat16)

            all_rdmas.extend(rs_rdmas)
            all_rdmas.extend(ag_rdmas)

        for rdma in all_rdmas:
            rdma.wait_send()

    return pl.pallas_call(
        body,
        out_shape=jax.ShapeDtypeStruct((b, d_shard), jnp.float32),
        in_specs=[pl.BlockSpec(memory_space=pltpu.VMEM)]
        + [pl.BlockSpec(memory_space=pl.ANY)] * 6,
        out_specs=pl.BlockSpec(memory_space=pltpu.VMEM),
        scratch_shapes=[
            pltpu.VMEM((N_LAYERS, N_DEV, b, blk), jnp.bfloat16),
            pltpu.VMEM((N_LAYERS, N_DEV - 1, b, blk), jnp.bfloat16),
            pltpu.VMEM((N_LAYERS, N_DEV, b, blk), jnp.bfloat16),
            pltpu.VMEM((N_LAYERS, Win0.shape[0], h_dim), jnp.float32),
            pltpu.VMEM((N_LAYERS, h_dim, d_shard), jnp.float32),
            pltpu.VMEM((N_LAYERS, Win0.shape[0], h_dim), jnp.bfloat16),
            pltpu.VMEM((N_LAYERS, h_dim, d_shard), jnp.bfloat16),
            pltpu.SemaphoreType.DMA((2, N_LAYERS)),
            pltpu.SemaphoreType.DMA((N_LAYERS, 2, N_DEV - 1)),
            pltpu.SemaphoreType.DMA((N_LAYERS, 2, N_DEV - 1)),
        ],
        compiler_params=pltpu.CompilerParams(collective_id=0),
    )(x, Win0, Wout0, Win1, Wout1, Win2, Wout2)


# device time: 36380 ns/iter; 1.0560x vs baseline; 1.0560x over previous
import jax
import jax.numpy as jnp
from jax import lax
from jax.experimental import pallas as pl
from jax.experimental.pallas import tpu as pltpu

N_DEV = 8
N_LAYERS = 3


def kernel(x, Win0, Wout0, Win1, Wout1, Win2, Wout2):
    b, d_shard = x.shape
    h_dim = Win0.shape[1]
    blk = h_dim // N_DEV

    def body(x_ref, win0_ref, wout0_ref, win1_ref, wout1_ref, win2_ref,
             wout2_ref, out_ref, part_ref, rs_ref, ag_ref,
             send_sems, recv_sems):
        my_pos = lax.axis_index("i")

        wins = [win0_ref, win1_ref, win2_ref]
        wouts = [wout0_ref, wout1_ref, wout2_ref]

        x_cur = x_ref[:, :]
        all_rdmas = []
        for l in range(N_LAYERS):
            partial = jnp.dot(x_cur, wins[l][:, :],
                              preferred_element_type=jnp.float32)
            for t in range(N_DEV):
                part_ref[l, t, :, :] = (
                    partial[:, t * blk:(t + 1) * blk].astype(jnp.bfloat16))

            if l == 0:
                barrier_sem = pltpu.get_barrier_semaphore()
                for d in range(1, N_DEV):
                    pl.semaphore_signal(
                        barrier_sem, inc=1,
                        device_id=((my_pos + d) % N_DEV,),
                        device_id_type=pl.DeviceIdType.MESH,
                    )
                pl.semaphore_wait(barrier_sem, N_DEV - 1)

            rs_rdmas = []
            for d in range(1, N_DEV):
                t = (my_pos + d) % N_DEV
                rdma = pltpu.make_async_remote_copy(
                    src_ref=part_ref.at[l, t],
                    dst_ref=rs_ref.at[l, d - 1],
                    send_sem=send_sems.at[l, 0, d - 1],
                    recv_sem=recv_sems.at[l, 0, d - 1],
                    device_id=(t,),
                    device_id_type=pl.DeviceIdType.MESH,
                )
                rdma.start()
                rs_rdmas.append(rdma)

            acc = part_ref[l, my_pos, :, :].astype(jnp.float32)
            for d in range(1, N_DEV):
                rs_rdmas[d - 1].wait_recv()
                acc = acc + rs_ref[l, d - 1, :, :].astype(jnp.float32)
            hred = jnp.maximum(acc, 0.0).astype(jnp.bfloat16)
            ag_ref[l, my_pos, :, :] = hred

            ag_rdmas = []
            for d in range(1, N_DEV):
                t = (my_pos + d) % N_DEV
                rdma = pltpu.make_async_remote_copy(
                    src_ref=ag_ref.at[l, my_pos],
                    dst_ref=ag_ref.at[l, my_pos],
                    send_sem=send_sems.at[l, 1, d - 1],
                    recv_sem=recv_sems.at[l, 1, d - 1],
                    device_id=(t,),
                    device_id_type=pl.DeviceIdType.MESH,
                )
                rdma.start()
                ag_rdmas.append(rdma)

            nxt = jnp.dot(
                ag_ref[l, my_pos, :, :],
                wouts[l][pl.ds(my_pos * blk, blk), :],
                preferred_element_type=jnp.float32)
            for d in range(1, N_DEV):
                ag_rdmas[d - 1].wait_recv()
                s = (my_pos - d) % N_DEV
                nxt = nxt + jnp.dot(
                    ag_ref[l, s, :, :],
                    wouts[l][pl.ds(s * blk, blk), :],
                    preferred_element_type=jnp.float32)
            if l == N_LAYERS - 1:
                out_ref[:, :] = nxt
            else:
                x_cur = nxt.astype(jnp.bfloat16)

            all_rdmas.extend(rs_rdmas)
            all_rdmas.extend(ag_rdmas)

        for rdma in all_rdmas:
            rdma.wait_send()

    pallas_fn = pl.pallas_call(
        body,
        out_shape=jax.ShapeDtypeStruct((b, d_shard), jnp.float32),
        in_specs=[pl.BlockSpec(memory_space=pltpu.VMEM)] * 7,
        out_specs=pl.BlockSpec(memory_space=pltpu.VMEM),
        scratch_shapes=[
            pltpu.VMEM((N_LAYERS, N_DEV, b, blk), jnp.bfloat16),
            pltpu.VMEM((N_LAYERS, N_DEV - 1, b, blk), jnp.bfloat16),
            pltpu.VMEM((N_LAYERS, N_DEV, b, blk), jnp.bfloat16),
            pltpu.SemaphoreType.DMA((N_LAYERS, 2, N_DEV - 1)),
            pltpu.SemaphoreType.DMA((N_LAYERS, 2, N_DEV - 1)),
        ],
        compiler_params=pltpu.CompilerParams(collective_id=0),
    )

    bf = jnp.bfloat16
    return pallas_fn(
        x.astype(bf), Win0.astype(bf), Wout0.astype(bf), Win1.astype(bf),
        Wout1.astype(bf), Win2.astype(bf), Wout2.astype(bf))


# device time: 29710 ns/iter; 1.2931x vs baseline; 1.2245x over previous
import jax
import jax.numpy as jnp
from jax import lax
from jax.experimental import pallas as pl
from jax.experimental.pallas import tpu as pltpu

N_DEV = 8
N_LAYERS = 3


def kernel(x, Win0, Wout0, Win1, Wout1, Win2, Wout2):
    b, d_shard = x.shape
    h_dim = Win0.shape[1]
    blk = h_dim // N_DEV

    def body(x_ref, win0_ref, wout0_ref, win1_ref, wout1_ref, win2_ref,
             wout2_ref, out_ref, part_ref, rs_ref, ag_ref,
             wf_in, wf_out, wbf_in, wbf_out, w_sems, send_sems, recv_sems):
        my_pos = lax.axis_index("i")

        win_hbm = [win0_ref, win1_ref, win2_ref]
        wout_hbm = [wout0_ref, wout1_ref, wout2_ref]

        win_dmas = [
            pltpu.make_async_copy(win_hbm[l], wf_in.at[l], w_sems.at[0, l])
            for l in range(N_LAYERS)
        ]
        wout_dmas = [
            pltpu.make_async_copy(wout_hbm[l], wf_out.at[l], w_sems.at[1, l])
            for l in range(N_LAYERS)
        ]
        for l in range(N_LAYERS):
            win_dmas[l].start()
            wout_dmas[l].start()

        win_dmas[0].wait()
        wbf_in[0, :, :] = wf_in[0, :, :].astype(jnp.bfloat16)
        x_cur = x_ref[:, :].astype(jnp.bfloat16)

        all_rdmas = []
        for l in range(N_LAYERS):
            partial = jnp.dot(x_cur, wbf_in[l, :, :],
                              preferred_element_type=jnp.float32)
            for t in range(N_DEV):
                part_ref[l, t, :, :] = (
                    partial[:, t * blk:(t + 1) * blk].astype(jnp.bfloat16))

            if l == 0:
                barrier_sem = pltpu.get_barrier_semaphore()
                for d in range(1, N_DEV):
                    pl.semaphore_signal(
                        barrier_sem, inc=1,
                        device_id=((my_pos + d) % N_DEV,),
                        device_id_type=pl.DeviceIdType.MESH,
                    )
                pl.semaphore_wait(barrier_sem, N_DEV - 1)

            rs_rdmas = []
            for d in range(1, N_DEV):
                t = (my_pos + d) % N_DEV
                rdma = pltpu.make_async_remote_copy(
                    src_ref=part_ref.at[l, t],
                    dst_ref=rs_ref.at[l, d - 1],
                    send_sem=send_sems.at[l, 0, d - 1],
                    recv_sem=recv_sems.at[l, 0, d - 1],
                    device_id=(t,),
                    device_id_type=pl.DeviceIdType.MESH,
                )
                rdma.start()
                rs_rdmas.append(rdma)

            wout_dmas[l].wait()
            wbf_out[l, :, :] = wf_out[l, :, :].astype(jnp.bfloat16)
            if l + 1 < N_LAYERS:
                win_dmas[l + 1].wait()
                wbf_in[l + 1, :, :] = wf_in[l + 1, :, :].astype(jnp.bfloat16)

            acc = part_ref[l, my_pos, :, :].astype(jnp.float32)
            for d in range(1, N_DEV):
                rs_rdmas[d - 1].wait_recv()
                acc = acc + rs_ref[l, d - 1, :, :].astype(jnp.float32)
            hred = jnp.maximum(acc, 0.0).astype(jnp.bfloat16)
            ag_ref[l, my_pos, :, :] = hred

            ag_rdmas = []
            for d in range(1, N_DEV):
                t = (my_pos + d) % N_DEV
                rdma = pltpu.make_async_remote_copy(
                    src_ref=ag_ref.at[l, my_pos],
                    dst_ref=ag_ref.at[l, my_pos],
                    send_sem=send_sems.at[l, 1, d - 1],
                    recv_sem=recv_sems.at[l, 1, d - 1],
                    device_id=(t,),
                    device_id_type=pl.DeviceIdType.MESH,
                )
                rdma.start()
                ag_rdmas.append(rdma)

            nxt = jnp.dot(
                ag_ref[l, my_pos, :, :],
                wbf_out[l, pl.ds(my_pos * blk, blk), :],
                preferred_element_type=jnp.float32)
            for d in range(1, N_DEV):
                ag_rdmas[d - 1].wait_recv()
                s = (my_pos - d) % N_DEV
                nxt = nxt + jnp.dot(
                    ag_ref[l, s, :, :],
                    wbf_out[l, pl.ds(s * blk, blk), :],
                    preferred_element_type=jnp.float32)
            if l == N_LAYERS - 1:
                out_ref[:, :] = nxt
            else:
                x_cur = nxt.astype(jnp.bfloat16)

            all_rdmas.extend(rs_rdmas)
            all_rdmas.extend(ag_rdmas)

        for rdma in all_rdmas:
            rdma.wait_send()

    hbm = pltpu.MemorySpace.HBM
    pallas_fn = pl.pallas_call(
        body,
        out_shape=jax.ShapeDtypeStruct((b, d_shard), jnp.float32),
        in_specs=[pl.BlockSpec(memory_space=pltpu.VMEM)]
        + [pl.BlockSpec(memory_space=hbm)] * 6,
        out_specs=pl.BlockSpec(memory_space=pltpu.VMEM),
        scratch_shapes=[
            pltpu.VMEM((N_LAYERS, N_DEV, b, blk), jnp.bfloat16),
            pltpu.VMEM((N_LAYERS, N_DEV - 1, b, blk), jnp.bfloat16),
            pltpu.VMEM((N_LAYERS, N_DEV, b, blk), jnp.bfloat16),
            pltpu.VMEM((N_LAYERS, Win0.shape[0], h_dim), jnp.float32),
            pltpu.VMEM((N_LAYERS, h_dim, d_shard), jnp.float32),
            pltpu.VMEM((N_LAYERS, Win0.shape[0], h_dim), jnp.bfloat16),
            pltpu.VMEM((N_LAYERS, h_dim, d_shard), jnp.bfloat16),
            pltpu.SemaphoreType.DMA((2, N_LAYERS)),
            pltpu.SemaphoreType.DMA((N_LAYERS, 2, N_DEV - 1)),
            pltpu.SemaphoreType.DMA((N_LAYERS, 2, N_DEV - 1)),
        ],
        compiler_params=pltpu.CompilerParams(collective_id=0),
    )

    c = lambda w: pltpu.with_memory_space_constraint(w, hbm)
    return pallas_fn(
        x, c(Win0), c(Wout0), c(Win1), c(Wout1), c(Win2), c(Wout2))


# device time: 26069 ns/iter; 1.4737x vs baseline; 1.1397x over previous
import jax
import jax.numpy as jnp
from jax import lax
from jax.experimental import pallas as pl
from jax.experimental.pallas import tpu as pltpu

N_DEV = 8
N_LAYERS = 3

SEND_ORDER = (6, 2, 5, 3, 1, 7, 4)
WAIT_ORDER = tuple(reversed(SEND_ORDER))


def kernel(x, Win0, Wout0, Win1, Wout1, Win2, Wout2):
    b, d_shard = x.shape
    h_dim = Win0.shape[1]
    blk = h_dim // N_DEV

    def body(x_ref, win0_ref, wout0_ref, win1_ref, wout1_ref, win2_ref,
             wout2_ref, out_ref, part_ref, rs_ref, ag_ref,
             wf_in, wf_out, wbf_in, wbf_out, w_sems, send_sems, recv_sems):
        my_pos = lax.axis_index("i")

        win_hbm = [win0_ref, win1_ref, win2_ref]
        wout_hbm = [wout0_ref, wout1_ref, wout2_ref]

        win_dmas = [
            pltpu.make_async_copy(win_hbm[l], wf_in.at[l], w_sems.at[0, l])
            for l in range(N_LAYERS)
        ]
        wout_dmas = [
            pltpu.make_async_copy(wout_hbm[l], wf_out.at[l], w_sems.at[1, l])
            for l in range(N_LAYERS)
        ]
        for l in range(N_LAYERS):
            win_dmas[l].start()
            wout_dmas[l].start()

        barrier_sem = pltpu.get_barrier_semaphore()
        for d in range(1, N_DEV):
            pl.semaphore_signal(
                barrier_sem, inc=1,
                device_id=((my_pos + d) % N_DEV,),
                device_id_type=pl.DeviceIdType.MESH,
            )

        win_dmas[0].wait()
        wbf_in[0, :, :] = wf_in[0, :, :].astype(jnp.bfloat16)
        x_cur = x_ref[:, :].astype(jnp.bfloat16)

        all_rdmas = []
        for l in range(N_LAYERS):
            partial = jnp.dot(x_cur, wbf_in[l, :, :],
                              preferred_element_type=jnp.float32)
            for t in range(N_DEV):
                part_ref[l, t, :, :] = (
                    partial[:, t * blk:(t + 1) * blk].astype(jnp.bfloat16))

            if l == 0:
                pl.semaphore_wait(barrier_sem, N_DEV - 1)

            rs_rdmas = {}
            for d in SEND_ORDER:
                t = (my_pos + d) % N_DEV
                rdma = pltpu.make_async_remote_copy(
                    src_ref=part_ref.at[l, t],
                    dst_ref=rs_ref.at[l, d - 1],
                    send_sem=send_sems.at[l, 0, d - 1],
                    recv_sem=recv_sems.at[l, 0, d - 1],
                    device_id=(t,),
                    device_id_type=pl.DeviceIdType.MESH,
                )
                rdma.start()
                rs_rdmas[d] = rdma

            wout_dmas[l].wait()
            wbf_out[l, :, :] = wf_out[l, :, :].astype(jnp.bfloat16)
            if l + 1 < N_LAYERS:
                win_dmas[l + 1].wait()
                wbf_in[l + 1, :, :] = wf_in[l + 1, :, :].astype(jnp.bfloat16)

            acc = part_ref[l, my_pos, :, :].astype(jnp.float32)
            for d in WAIT_ORDER:
                rs_rdmas[d].wait_recv()
                acc = acc + rs_ref[l, d - 1, :, :].astype(jnp.float32)
            hred = jnp.maximum(acc, 0.0).astype(jnp.bfloat16)
            ag_ref[l, my_pos, :, :] = hred

            ag_rdmas = {}
            for d in SEND_ORDER:
                t = (my_pos + d) % N_DEV
                rdma = pltpu.make_async_remote_copy(
                    src_ref=ag_ref.at[l, my_pos],
                    dst_ref=ag_ref.at[l, my_pos],
                    send_sem=send_sems.at[l, 1, d - 1],
                    recv_sem=recv_sems.at[l, 1, d - 1],
                    device_id=(t,),
                    device_id_type=pl.DeviceIdType.MESH,
                )
                rdma.start()
                ag_rdmas[d] = rdma

            nxt = jnp.dot(
                ag_ref[l, my_pos, :, :],
                wbf_out[l, pl.ds(my_pos * blk, blk), :],
                preferred_element_type=jnp.float32)
            for d in WAIT_ORDER:
                ag_rdmas[d].wait_recv()
                s = (my_pos - d) % N_DEV
                nxt = nxt + jnp.dot(
                    ag_ref[l, s, :, :],
                    wbf_out[l, pl.ds(s * blk, blk), :],
                    preferred_element_type=jnp.float32)
            if l == N_LAYERS - 1:
                out_ref[:, :] = nxt
            else:
                x_cur = nxt.astype(jnp.bfloat16)

            all_rdmas.extend(rs_rdmas.values())
            all_rdmas.extend(ag_rdmas.values())

        for rdma in all_rdmas:
            rdma.wait_send()

    hbm = pltpu.MemorySpace.HBM
    pallas_fn = pl.pallas_call(
        body,
        out_shape=jax.ShapeDtypeStruct((b, d_shard), jnp.float32),
        in_specs=[pl.BlockSpec(memory_space=pltpu.VMEM)]
        + [pl.BlockSpec(memory_space=hbm)] * 6,
        out_specs=pl.BlockSpec(memory_space=pltpu.VMEM),
        scratch_shapes=[
            pltpu.VMEM((N_LAYERS, N_DEV, b, blk), jnp.bfloat16),
            pltpu.VMEM((N_LAYERS, N_DEV - 1, b, blk), jnp.bfloat16),
            pltpu.VMEM((N_LAYERS, N_DEV, b, blk), jnp.bfloat16),
            pltpu.VMEM((N_LAYERS, Win0.shape[0], h_dim), jnp.float32),
            pltpu.VMEM((N_LAYERS, h_dim, d_shard), jnp.float32),
            pltpu.VMEM((N_LAYERS, Win0.shape[0], h_dim), jnp.bfloat16),
            pltpu.VMEM((N_LAYERS, h_dim, d_shard), jnp.bfloat16),
            pltpu.SemaphoreType.DMA((2, N_LAYERS)),
            pltpu.SemaphoreType.DMA((N_LAYERS, 2, N_DEV - 1)),
            pltpu.SemaphoreType.DMA((N_LAYERS, 2, N_DEV - 1)),
        ],
        compiler_params=pltpu.CompilerParams(collective_id=0),
    )

    c = lambda w: pltpu.with_memory_space_constraint(w, hbm)
    return pallas_fn(
        x, c(Win0), c(Wout0), c(Win1), c(Wout1), c(Win2), c(Wout2))
